# baseline (speedup 1.0000x reference)
import functools

import jax
import jax.numpy as jnp
from jax import lax
from jax.experimental import pallas as pl
from jax.experimental.pallas import tpu as pltpu
from jax.experimental.pallas import tpu_sc as plsc

N = 10000
E = 320000
D_IN = 128
D_HID = 512

NC, NS = 2, 16
NW = NC * NS
CH = 128
CPT = 80
EPT = CH * CPT
E_PAD = NW * EPT
N_PAD = 10016
N_ACC = 10240
N_OFF = 10400
RPT = N_ACC // NS
ZC = RPT // CH

_sc_mesh = plsc.VectorSubcoreMesh(core_axis_name="c", subcore_axis_name="s")


def _agg_body(u_hbm, sd_hbm, zeros_hbm, out_hbm,
              idx_v, bufa, bufb, acc_sh,
              sem_ia, sem_ib, sem_ga, sem_gb, sem_sa, sem_sb):
    c = lax.axis_index("c")
    s = lax.axis_index("s")
    tid = c * NS + s

    bufs = (bufa, bufb)
    isems = (sem_ia, sem_ib)
    gsems = (sem_ga, sem_gb)
    ssems = (sem_sa, sem_sb)

    def idx_start(k, b):
        pltpu.async_copy(sd_hbm.at[tid * CPT + k], idx_v.at[b], isems[b])

    def idx_wait(b):
        pltpu.make_async_copy(sd_hbm.at[0], idx_v.at[b], isems[b]).wait()

    def gather_start(b):
        pltpu.async_copy(u_hbm.at[idx_v.at[b, 0]], bufs[b], gsems[b])

    def gather_wait(b):
        pltpu.make_async_copy(u_hbm.at[idx_v.at[b, 0]], bufs[b],
                              gsems[b]).wait()

    def scatter_start(b):
        pltpu.async_copy(bufs[b], acc_sh.at[idx_v.at[b, 1]], ssems[b],
                         add=True)

    def scatter_wait(b):
        pltpu.make_async_copy(bufs[b], acc_sh.at[idx_v.at[b, 1]],
                              ssems[b]).wait()

    idx_start(0, 0)
    idx_start(1, 1)
    idx_wait(0)
    gather_start(0)
    for k in range(ZC):
        pltpu.sync_copy(zeros_hbm, acc_sh.at[pl.ds((s * ZC + k) * CH, CH)])
    plsc.subcore_barrier()

    def half(k, b):
        gather_wait(b)
        idx_wait(1 - b)
        gather_start(1 - b)
        scatter_start(b)
        scatter_wait(b)
        idx_start(k + 2, b)

    def loop_body(i, carry):
        for b in range(2):
            half(i * 2 + b, b)
        return carry

    lax.fori_loop(0, (CPT - 2) // 2, loop_body, 0)

    gather_wait(0)
    idx_wait(1)
    gather_start(1)
    scatter_start(0)
    scatter_wait(0)
    gather_wait(1)
    scatter_start(1)
    scatter_wait(1)

    plsc.subcore_barrier()

    for k in range(ZC):
        row = (s * ZC + k) * CH
        pltpu.sync_copy(acc_sh.at[pl.ds(row, CH)],
                        out_hbm.at[pl.ds(c * N_OFF + row, CH)])


_sc_agg = functools.partial(
    pl.kernel,
    out_type=jax.ShapeDtypeStruct((N_OFF + N_ACC, D_IN), jnp.float32),
    mesh=_sc_mesh,
    scratch_types=[
        pltpu.VMEM((2, 2, CH), jnp.int32),
        pltpu.VMEM((CH, D_IN), jnp.float32),
        pltpu.VMEM((CH, D_IN), jnp.float32),
        pltpu.VMEM_SHARED((N_ACC, D_IN), jnp.float32),
        pltpu.SemaphoreType.DMA,
        pltpu.SemaphoreType.DMA,
        pltpu.SemaphoreType.DMA,
        pltpu.SemaphoreType.DMA,
        pltpu.SemaphoreType.DMA,
        pltpu.SemaphoreType.DMA,
    ],
)(_agg_body)


MB = 400
GRID = N // MB
POFF = N_OFF // MB


def _pre_body(d0_ref, d1_ref, x_ref, u_ref, dv_ref):
    deg = d0_ref[:, :1] + d1_ref[:, :1] + 1.0
    dinv = lax.rsqrt(deg)
    u_ref[...] = x_ref[...] * dinv
    dv_ref[...] = jnp.broadcast_to(dinv, dv_ref.shape)


def _mm_pair_body(dv_ref, p0_ref, p1_ref, u_ref, wa_ref, ba_ref, wb_ref,
                  o_ref):
    dinv = dv_ref[...]
    a = (p0_ref[...] + p1_ref[...] + u_ref[...]) * dinv
    h = jnp.maximum(jnp.dot(a, wa_ref[...]) + ba_ref[...], 0.0)
    o_ref[...] = jnp.dot(h, wb_ref[...]) * dinv


def _z_body(dv_ref, p0_ref, p1_ref, u_ref, b_ref, z_ref, u3_ref):
    dinv = dv_ref[...]
    z = (p0_ref[...] + p1_ref[...] + u_ref[...]) * dinv + b_ref[...]
    z_ref[...] = z
    u3_ref[...] = z * dinv


def _final_body(dv_ref, p0_ref, p1_ref, u_ref, b_ref, o_ref):
    o_ref[...] = ((p0_ref[...] + p1_ref[...] + u_ref[...]) * dv_ref[...]
                  + b_ref[...])


def _row_spec(w):
    return pl.BlockSpec((MB, w), lambda i: (i, 0))


def _off_spec(w):
    return pl.BlockSpec((MB, w), lambda i: (POFF + i, 0))


def _full_spec(r, c):
    return pl.BlockSpec((r, c), lambda i: (0, 0))


def _tc_pre(dparts, x):
    return pl.pallas_call(
        _pre_body,
        grid=(GRID,),
        in_specs=[_row_spec(D_IN), _off_spec(D_IN), _row_spec(D_IN)],
        out_specs=[_row_spec(D_IN), _row_spec(D_IN)],
        out_shape=[jax.ShapeDtypeStruct((N_PAD, D_IN), jnp.float32),
                   jax.ShapeDtypeStruct((N, D_IN), jnp.float32)],
    )(dparts, dparts, x)


def _tc_mm_pair(dv, parts, u, wa, ba, wb):
    return pl.pallas_call(
        _mm_pair_body,
        grid=(GRID,),
        in_specs=[_row_spec(D_IN), _row_spec(D_IN), _off_spec(D_IN),
                  _row_spec(D_IN),
                  _full_spec(D_IN, D_HID), _full_spec(1, D_HID),
                  _full_spec(D_HID, D_IN)],
        out_specs=_row_spec(D_IN),
        out_shape=jax.ShapeDtypeStruct((N_PAD, D_IN), jnp.float32),
    )(dv, parts, parts, u, wa, ba, wb)


def _tc_z(dv, parts, u, b):
    return pl.pallas_call(
        _z_body,
        grid=(GRID,),
        in_specs=[_row_spec(D_IN), _row_spec(D_IN), _off_spec(D_IN),
                  _row_spec(D_IN), _full_spec(1, D_IN)],
        out_specs=[_row_spec(D_IN), _row_spec(D_IN)],
        out_shape=[jax.ShapeDtypeStruct((N, D_IN), jnp.float32),
                   jax.ShapeDtypeStruct((N_PAD, D_IN), jnp.float32)],
    )(dv, parts, parts, u, b)


def _tc_final(dv, parts, u, b):
    return pl.pallas_call(
        _final_body,
        grid=(GRID,),
        in_specs=[_row_spec(D_IN), _row_spec(D_IN), _off_spec(D_IN),
                  _row_spec(D_IN), _full_spec(1, D_IN)],
        out_specs=_row_spec(D_IN),
        out_shape=jax.ShapeDtypeStruct((N, D_IN), jnp.float32),
    )(dv, parts, parts, u, b)


@jax.jit
def kernel(x, edge_index, W1, b1, W2, b2, W3, b3, W4, b4):
    pad = E_PAD - E
    src = jnp.concatenate(
        [edge_index[0], jnp.int32(N) + (jnp.arange(pad, dtype=jnp.int32) % 16)])
    dst = jnp.concatenate(
        [edge_index[1],
         jnp.int32(N) + (jnp.arange(pad, dtype=jnp.int32) % (N_ACC - N))])
    src_r = src.reshape(NW * CPT, CH)
    dst_r = dst.reshape(NW * CPT, CH)
    sd = jnp.stack([src_r, dst_r], axis=1)
    sd_h = jnp.stack([jnp.zeros_like(dst_r), dst_r], axis=1)
    zeros128 = jnp.zeros((CH, D_IN), jnp.float32)
    b1r = b1.reshape(1, D_HID)
    b2r = b2.reshape(1, D_IN)
    b3r = b3.reshape(1, D_HID)
    b4r = b4.reshape(1, D_IN)

    ones_pad = jnp.ones((N_PAD, D_IN), jnp.float32)
    dparts = _sc_agg(ones_pad, sd_h, zeros128)

    u1, dv = _tc_pre(dparts, x)
    parts = _sc_agg(u1, sd, zeros128)
    u2 = _tc_mm_pair(dv, parts, u1, W1, b1r, W2)

    parts = _sc_agg(u2, sd, zeros128)
    z, u3 = _tc_z(dv, parts, u2, b2r)

    parts = _sc_agg(u3, sd, zeros128)
    u4 = _tc_mm_pair(dv, parts, u3, W3, b3r, W4)

    parts = _sc_agg(u4, sd, zeros128)
    x_recon = _tc_final(dv, parts, u4, b4r)

    return (x_recon, z)


# speedup vs baseline: 16.4940x; 16.4940x over previous
import functools

import jax
import jax.numpy as jnp
from jax import lax
from jax.experimental import pallas as pl
from jax.experimental.pallas import tpu as pltpu
from jax.experimental.pallas import tpu_sc as plsc

N = 10000
E = 320000
D_IN = 128
D_HID = 512

NC, NS = 2, 16
NW = NC * NS
CH = 128
EPT = E // NW
CPT = EPT // CH
TL = EPT - CPT * CH
N_ACC = 10240
N_OFF = 10400
RPT = N_ACC // NS
ZC = RPT // CH

_sc_mesh = plsc.VectorSubcoreMesh(core_axis_name="c", subcore_axis_name="s")


def _agg_body(u_hbm, ei_hbm, zeros_hbm, out_hbm,
              idx_v, tidx_v, bufa, bufb, acc_sh,
              sem_ia, sem_ib, sem_ga, sem_gb, sem_sa, sem_sb):
    c = lax.axis_index("c")
    s = lax.axis_index("s")
    tid = c * NS + s
    base = tid * EPT

    bufs = (bufa, bufb)
    isems = (sem_ia, sem_ib)
    gsems = (sem_ga, sem_gb)
    ssems = (sem_sa, sem_sb)

    def idx_start(k, b):
        pltpu.async_copy(ei_hbm.at[pl.ds(base + k * CH, CH)],
                         idx_v.at[b, 0], isems[b])
        pltpu.async_copy(ei_hbm.at[pl.ds(E + base + k * CH, CH)],
                         idx_v.at[b, 1], isems[b])

    def idx_wait(b):
        pltpu.make_async_copy(ei_hbm.at[pl.ds(0, CH)], idx_v.at[b, 0],
                              isems[b]).wait()
        pltpu.make_async_copy(ei_hbm.at[pl.ds(0, CH)], idx_v.at[b, 1],
                              isems[b]).wait()

    def gather_start(b):
        pltpu.async_copy(u_hbm.at[idx_v.at[b, 0]], bufs[b], gsems[b])

    def gather_wait(b):
        pltpu.make_async_copy(u_hbm.at[idx_v.at[b, 0]], bufs[b],
                              gsems[b]).wait()

    def scatter_start(b):
        pltpu.async_copy(bufs[b], acc_sh.at[idx_v.at[b, 1]], ssems[b],
                         add=True)

    def scatter_wait(b):
        pltpu.make_async_copy(bufs[b], acc_sh.at[idx_v.at[b, 1]],
                              ssems[b]).wait()

    idx_start(0, 0)
    idx_start(1, 1)
    idx_wait(0)
    gather_start(0)
    for k in range(ZC):
        pltpu.sync_copy(zeros_hbm, acc_sh.at[pl.ds((s * ZC + k) * CH, CH)])
    plsc.subcore_barrier()

    def half(k, b):
        gather_wait(b)
        idx_wait(1 - b)
        gather_start(1 - b)
        scatter_start(b)
        scatter_wait(b)
        idx_start(k + 2, b)

    def loop_body(i, carry):
        for b in range(2):
            half(i * 2 + b, b)
        return carry

    lax.fori_loop(0, (CPT - 2) // 2, loop_body, 0)

    pltpu.async_copy(ei_hbm.at[pl.ds(base + CPT * CH, TL)],
                     tidx_v.at[0], sem_ia)
    pltpu.async_copy(ei_hbm.at[pl.ds(E + base + CPT * CH, TL)],
                     tidx_v.at[1], sem_ia)
    gather_wait(0)
    idx_wait(1)
    gather_start(1)
    scatter_start(0)
    scatter_wait(0)
    gather_wait(1)
    scatter_start(1)
    scatter_wait(1)
    pltpu.make_async_copy(ei_hbm.at[pl.ds(0, TL)], tidx_v.at[0],
                          sem_ia).wait()
    pltpu.make_async_copy(ei_hbm.at[pl.ds(0, TL)], tidx_v.at[1],
                          sem_ia).wait()
    tbuf = bufa.at[pl.ds(0, TL)]
    pltpu.async_copy(u_hbm.at[tidx_v.at[0]], tbuf, sem_ga)
    pltpu.make_async_copy(u_hbm.at[tidx_v.at[0]], tbuf, sem_ga).wait()
    pltpu.async_copy(tbuf, acc_sh.at[tidx_v.at[1]], sem_sa, add=True)
    pltpu.make_async_copy(tbuf, acc_sh.at[tidx_v.at[1]], sem_sa).wait()

    plsc.subcore_barrier()

    for k in range(ZC):
        row = (s * ZC + k) * CH
        pltpu.sync_copy(acc_sh.at[pl.ds(row, CH)],
                        out_hbm.at[pl.ds(c * N_OFF + row, CH)])


_sc_agg = functools.partial(
    pl.kernel,
    out_type=jax.ShapeDtypeStruct((N_OFF + N_ACC, D_IN), jnp.float32),
    mesh=_sc_mesh,
    scratch_types=[
        pltpu.VMEM((2, 2, CH), jnp.int32),
        pltpu.VMEM((2, TL), jnp.int32),
        pltpu.VMEM((CH, D_IN), jnp.float32),
        pltpu.VMEM((CH, D_IN), jnp.float32),
        pltpu.VMEM_SHARED((N_ACC, D_IN), jnp.float32),
        pltpu.SemaphoreType.DMA,
        pltpu.SemaphoreType.DMA,
        pltpu.SemaphoreType.DMA,
        pltpu.SemaphoreType.DMA,
        pltpu.SemaphoreType.DMA,
        pltpu.SemaphoreType.DMA,
    ],
)(_agg_body)


MB = 400
GRID = N // MB
POFF = N_OFF // MB


def _pre_body(d0_ref, d1_ref, x_ref, u_ref, dv_ref):
    deg = d0_ref[:, :1] + d1_ref[:, :1] + 1.0
    dinv = lax.rsqrt(deg)
    u_ref[...] = x_ref[...] * dinv
    dv_ref[...] = jnp.broadcast_to(dinv, dv_ref.shape)


def _mm_pair_body(dv_ref, p0_ref, p1_ref, u_ref, wa_ref, ba_ref, wb_ref,
                  o_ref):
    dinv = dv_ref[...]
    a = (p0_ref[...] + p1_ref[...] + u_ref[...]) * dinv
    h = jnp.maximum(jnp.dot(a, wa_ref[...]) + ba_ref[...], 0.0)
    o_ref[...] = jnp.dot(h, wb_ref[...]) * dinv


def _z_body(dv_ref, p0_ref, p1_ref, u_ref, b_ref, z_ref, u3_ref):
    dinv = dv_ref[...]
    z = (p0_ref[...] + p1_ref[...] + u_ref[...]) * dinv + b_ref[...]
    z_ref[...] = z
    u3_ref[...] = z * dinv


def _final_body(dv_ref, p0_ref, p1_ref, u_ref, b_ref, o_ref):
    o_ref[...] = ((p0_ref[...] + p1_ref[...] + u_ref[...]) * dv_ref[...]
                  + b_ref[...])


def _row_spec(w):
    return pl.BlockSpec((MB, w), lambda i: (i, 0))


def _off_spec(w):
    return pl.BlockSpec((MB, w), lambda i: (POFF + i, 0))


def _full_spec(r, c):
    return pl.BlockSpec((r, c), lambda i: (0, 0))


def _tc_pre(dparts, x):
    return pl.pallas_call(
        _pre_body,
        grid=(GRID,),
        in_specs=[_row_spec(D_IN), _off_spec(D_IN), _row_spec(D_IN)],
        out_specs=[_row_spec(D_IN), _row_spec(D_IN)],
        out_shape=[jax.ShapeDtypeStruct((N, D_IN), jnp.float32),
                   jax.ShapeDtypeStruct((N, D_IN), jnp.float32)],
    )(dparts, dparts, x)


def _tc_mm_pair(dv, parts, u, wa, ba, wb):
    return pl.pallas_call(
        _mm_pair_body,
        grid=(GRID,),
        in_specs=[_row_spec(D_IN), _row_spec(D_IN), _off_spec(D_IN),
                  _row_spec(D_IN),
                  _full_spec(D_IN, D_HID), _full_spec(1, D_HID),
                  _full_spec(D_HID, D_IN)],
        out_specs=_row_spec(D_IN),
        out_shape=jax.ShapeDtypeStruct((N, D_IN), jnp.float32),
    )(dv, parts, parts, u, wa, ba, wb)


def _tc_z(dv, parts, u, b):
    return pl.pallas_call(
        _z_body,
        grid=(GRID,),
        in_specs=[_row_spec(D_IN), _row_spec(D_IN), _off_spec(D_IN),
                  _row_spec(D_IN), _full_spec(1, D_IN)],
        out_specs=[_row_spec(D_IN), _row_spec(D_IN)],
        out_shape=[jax.ShapeDtypeStruct((N, D_IN), jnp.float32),
                   jax.ShapeDtypeStruct((N, D_IN), jnp.float32)],
    )(dv, parts, parts, u, b)


def _tc_final(dv, parts, u, b):
    return pl.pallas_call(
        _final_body,
        grid=(GRID,),
        in_specs=[_row_spec(D_IN), _row_spec(D_IN), _off_spec(D_IN),
                  _row_spec(D_IN), _full_spec(1, D_IN)],
        out_specs=_row_spec(D_IN),
        out_shape=jax.ShapeDtypeStruct((N, D_IN), jnp.float32),
    )(dv, parts, parts, u, b)


@jax.jit
def kernel(x, edge_index, W1, b1, W2, b2, W3, b3, W4, b4):
    ei = edge_index.reshape(2 * E)
    zeros128 = jnp.zeros((CH, D_IN), jnp.float32)
    b1r = b1.reshape(1, D_HID)
    b2r = b2.reshape(1, D_IN)
    b3r = b3.reshape(1, D_HID)
    b4r = b4.reshape(1, D_IN)

    ones_pad = jnp.ones((N, D_IN), jnp.float32)
    dparts = _sc_agg(ones_pad, ei, zeros128)

    u1, dv = _tc_pre(dparts, x)
    parts = _sc_agg(u1, ei, zeros128)
    u2 = _tc_mm_pair(dv, parts, u1, W1, b1r, W2)

    parts = _sc_agg(u2, ei, zeros128)
    z, u3 = _tc_z(dv, parts, u2, b2r)

    parts = _sc_agg(u3, ei, zeros128)
    u4 = _tc_mm_pair(dv, parts, u3, W3, b3r, W4)

    parts = _sc_agg(u4, ei, zeros128)
    x_recon = _tc_final(dv, parts, u4, b4r)

    return (x_recon, z)
